# baseline (speedup 1.0000x reference)
import functools

import jax
import jax.numpy as jnp
from jax.experimental import pallas as pl
from jax.experimental.pallas import tpu as pltpu

N_NODES = 10000
N_EDGES = 160000
FEAT = 512
HID = 128
BUF = 4096
K = 16
BATCH = 20
NCLS = 4


def _fc_body(x_ref, w_ref, b_ref, o_ref):
    o_ref[...] = jax.nn.relu(
        jnp.dot(x_ref[...], w_ref[...], preferred_element_type=jnp.float32)
        + b_ref[...]
    )


def fc_relu(x, w, b):
    n = x.shape[0]
    pad = (-n) % 8
    xp = jnp.pad(x, ((0, pad), (0, 0)))
    out = pl.pallas_call(
        _fc_body,
        out_shape=jax.ShapeDtypeStruct((n + pad, w.shape[1]), jnp.float32),
        grid=(1,),
        in_specs=[
            pl.BlockSpec((n + pad, x.shape[1]), lambda i: (0, 0)),
            pl.BlockSpec((w.shape[0], w.shape[1]), lambda i: (0, 0)),
            pl.BlockSpec((w.shape[1],), lambda i: (0,)),
        ],
        out_specs=pl.BlockSpec((n + pad, w.shape[1]), lambda i: (0, 0)),
    )(xp, w, b)
    return out[:n]


def layer_norm(x, g, b, eps=1e-5):
    m = x.mean(-1, keepdims=True)
    v = ((x - m) ** 2).mean(-1, keepdims=True)
    return (x - m) / jnp.sqrt(v + eps) * g + b


def seg_softmax(vals, seg, num):
    m = jax.ops.segment_max(vals, seg, num)
    m = jnp.where(jnp.isneginf(m), 0.0, m)
    e = jnp.exp(vals - m[seg])
    s = jax.ops.segment_sum(e, seg, num)
    return e / (s[seg] + 1e-16)


def genconv(x, src, dst, c):
    msg = jax.nn.relu(x[src]) + 1e-7
    alpha = seg_softmax(msg * c['t'], dst, x.shape[0])
    out = jax.ops.segment_sum(msg * alpha, dst, x.shape[0])
    out = out + x
    h = out @ c['W1'] + c['b1']
    h = layer_norm(h, c['g1'], c['bt1'])
    h = jax.nn.relu(h)
    return h @ c['W2'] + c['b2']


def hypergraph_conv(x, node_idx, he_idx, he_attr, W, att, bias, num_nodes, num_he):
    Xl = x @ W
    He = he_attr @ W
    a = jnp.concatenate([Xl[node_idx], He[he_idx]], axis=-1)
    alpha = (a * att).sum(-1)
    alpha = jax.nn.leaky_relu(alpha, 0.2)
    alpha = seg_softmax(alpha, node_idx, num_nodes)
    ones = jnp.ones_like(alpha)
    Dn = jax.ops.segment_sum(ones, node_idx, num_nodes)
    Dn = jnp.where(Dn > 0, 1.0 / Dn, 0.0)
    Be = jax.ops.segment_sum(ones, he_idx, num_he)
    Be = jnp.where(Be > 0, 1.0 / Be, 0.0)
    m1 = Be[he_idx][:, None] * alpha[:, None] * Xl[node_idx]
    out_e = jax.ops.segment_sum(m1, he_idx, num_he)
    m2 = Dn[node_idx][:, None] * alpha[:, None] * out_e[he_idx]
    out = jax.ops.segment_sum(m2, node_idx, num_nodes)
    return out + bias


def graph_norm(x, w, b, ms, eps=1e-5):
    mean = x.mean(0, keepdims=True)
    out = x - ms * mean
    var = (out ** 2).mean(0, keepdims=True)
    return out / jnp.sqrt(var + eps) * w + b


def kernel(x, edge_index, edge_latent, y, params):
    p = params
    src, dst = edge_index[0], edge_index[1]
    x = fc_relu(x, p['W_fc'], p['b_fc'])
    x_ = x
    x = genconv(x_, src, dst, p['convs'][0])
    x_ = jnp.concatenate([x_, x], axis=-1)
    for i in (1, 2):
        h = genconv(x, src, dst, p['convs'][i])
        h = layer_norm(h, p['lns'][i - 1]['g'], p['lns'][i - 1]['b'])
        h = jax.nn.relu(h)
        x = x + h
        x_ = jnp.concatenate([x_, x], axis=-1)
    h_path = x_.reshape(BATCH, 500, 4 * HID)
    h_path = jax.nn.relu(h_path @ p['Wphi'] + p['bphi'])
    a = jnp.tanh(h_path @ p['Wa'] + p['ba'])
    bgate = jax.nn.sigmoid(h_path @ p['Wb'] + p['bb'])
    A = (a * bgate) @ p['Wc'] + p['bc']
    A = jnp.swapaxes(A, -1, -2)
    h_path = jax.nn.softmax(A, axis=-1) @ h_path
    h = jax.nn.relu(h_path @ p['Wrho'] + p['brho'])[:, 0, :]
    logits = h @ p['Wcls'] + p['bcls']
    x_concat = jnp.concatenate([h, p['rehearsal']], axis=0)[:BUF]
    sim = x_concat @ x_concat.T
    _, nbr = jax.lax.top_k(sim, K)
    node_idx = nbr.reshape(-1)
    he_idx = jnp.repeat(jnp.arange(BUF), K)
    ea = x_concat
    g = p['gcn']
    _x = hypergraph_conv(x_concat, node_idx, he_idx, ea, g['Whg1'], g['att1'], g['bhg1'], BUF, BUF)
    _x = graph_norm(_x, g['gw1'], g['gb1'], g['gm1'])
    _x = jax.nn.leaky_relu(_x, 0.01)
    out1 = jax.nn.leaky_relu(_x @ g['Wfc1'] + g['bfc1'], 0.01)
    _x = hypergraph_conv(_x, node_idx, he_idx, ea, g['Whg2'], g['att2'], g['bhg2'], BUF, BUF)
    _x = graph_norm(_x, g['gw2'], g['gb2'], g['gm2'])
    _x = jax.nn.leaky_relu(_x, 0.01)
    out2 = jax.nn.leaky_relu(_x @ g['Wfc2'] + g['bfc2'], 0.01)
    out = jnp.concatenate([x_concat, out1, out2], axis=1)
    s = jax.nn.relu(out.T @ g['Wat1'] + g['bat1']) @ g['Wat2'] + g['bat2']
    s = jax.nn.sigmoid(s)
    s = s[:, 0] - jnp.mean(s)
    H = out * s[None, :]
    H = H[:BATCH]
    H = jax.nn.leaky_relu(H @ g['Wc1'] + g['bc1'], 0.01)
    distil = H @ g['Wd'] + g['bd']
    H = jax.nn.leaky_relu(H @ g['Wc2'] + g['bc2'] + H, 0.01)
    logits_graph = H @ g['Wch'] + g['bch']
    return logits, logits_graph, distil


# speedup vs baseline: 2.2362x; 2.2362x over previous
import functools

import jax
import jax.numpy as jnp
from jax.experimental import pallas as pl
from jax.experimental.pallas import tpu as pltpu

N_NODES = 10000
N_EDGES = 160000
FEAT = 512
HID = 128
BUF = 4096
K = 16
BATCH = 20
NCLS = 4
NEG = -3.0e38


def _fc_body(x_ref, w_ref, b_ref, o_ref):
    o_ref[...] = jax.nn.relu(
        jnp.dot(x_ref[...], w_ref[...], preferred_element_type=jnp.float32)
        + b_ref[...]
    )


def fc_relu(x, w, b):
    n = x.shape[0]
    pad = (-n) % 8
    xp = jnp.pad(x, ((0, pad), (0, 0)))
    out = pl.pallas_call(
        _fc_body,
        out_shape=jax.ShapeDtypeStruct((n + pad, w.shape[1]), jnp.float32),
        grid=(1,),
        in_specs=[
            pl.BlockSpec((n + pad, x.shape[1]), lambda i: (0, 0)),
            pl.BlockSpec((w.shape[0], w.shape[1]), lambda i: (0, 0)),
            pl.BlockSpec((w.shape[1],), lambda i: (0,)),
        ],
        out_specs=pl.BlockSpec((n + pad, w.shape[1]), lambda i: (0, 0)),
    )(xp, w, b)
    return out[:n]


_TKR = 128


def _topk_body(xb_ref, xcT_ref, m_ref, sim_ref):
    sim_ref[...] = jnp.dot(
        xb_ref[...], xcT_ref[...], preferred_element_type=jnp.float32
    )
    m_ref[...] = jnp.zeros_like(m_ref)
    cols = jax.lax.broadcasted_iota(jnp.int32, (_TKR, BUF), 1)

    def step(t, carry):
        s = sim_ref[...]
        rmax = jnp.max(s, axis=1, keepdims=True)
        pick = jnp.min(
            jnp.where(s == rmax, cols, BUF), axis=1, keepdims=True
        )
        hit = cols == pick
        m_ref[...] += hit.astype(jnp.float32)
        sim_ref[...] = jnp.where(hit, NEG, s)
        return carry

    jax.lax.fori_loop(0, K, step, 0)


def topk_mask(xc):
    xcT = xc.T
    return pl.pallas_call(
        _topk_body,
        out_shape=jax.ShapeDtypeStruct((BUF, BUF), jnp.float32),
        grid=(BUF // _TKR,),
        in_specs=[
            pl.BlockSpec((_TKR, FEAT), lambda i: (i, 0)),
            pl.BlockSpec((FEAT, BUF), lambda i: (0, 0)),
        ],
        out_specs=pl.BlockSpec((_TKR, BUF), lambda i: (i, 0)),
        scratch_shapes=[pltpu.VMEM((_TKR, BUF), jnp.float32)],
    )(xc, xcT)


def _proj_he_body(x_ref, ea_ref, w_ref, aA_ref, aB_ref, xl_ref, u_ref, v_ref):
    xl = jnp.dot(x_ref[...], w_ref[...], preferred_element_type=jnp.float32)
    xl_ref[...] = xl
    u_ref[...] = jnp.sum(xl * aA_ref[...][None, :], axis=1)
    he = jnp.dot(ea_ref[...], w_ref[...], preferred_element_type=jnp.float32)
    v_ref[...] = jnp.sum(he * aB_ref[...][None, :], axis=1)


def _proj_same_body(x_ref, w_ref, aA_ref, aB_ref, xl_ref, u_ref, v_ref):
    xl = jnp.dot(x_ref[...], w_ref[...], preferred_element_type=jnp.float32)
    xl_ref[...] = xl
    u_ref[...] = jnp.sum(xl * aA_ref[...][None, :], axis=1)
    v_ref[...] = jnp.sum(xl * aB_ref[...][None, :], axis=1)


_PB = 512


def hyper_proj(x, W, att, ea=None):
    attA, attB = att[:FEAT], att[FEAT:]
    out_shape = [
        jax.ShapeDtypeStruct((BUF, FEAT), jnp.float32),
        jax.ShapeDtypeStruct((BUF,), jnp.float32),
        jax.ShapeDtypeStruct((BUF,), jnp.float32),
    ]
    out_specs = [
        pl.BlockSpec((_PB, FEAT), lambda i: (i, 0)),
        pl.BlockSpec((_PB,), lambda i: (i,)),
        pl.BlockSpec((_PB,), lambda i: (i,)),
    ]
    wspec = pl.BlockSpec((FEAT, FEAT), lambda i: (0, 0))
    aspec = pl.BlockSpec((FEAT,), lambda i: (0,))
    xspec = pl.BlockSpec((_PB, FEAT), lambda i: (i, 0))
    if ea is None:
        return pl.pallas_call(
            _proj_same_body,
            out_shape=out_shape,
            grid=(BUF // _PB,),
            in_specs=[xspec, wspec, aspec, aspec],
            out_specs=out_specs,
        )(x, W, attA, attB)
    return pl.pallas_call(
        _proj_he_body,
        out_shape=out_shape,
        grid=(BUF // _PB,),
        in_specs=[xspec, xspec, wspec, aspec, aspec],
        out_specs=out_specs,
    )(x, ea, W, attA, attB)


_CB = 512


def _colsm_body(m_ref, u_ref, v_ref, a_ref, dn_ref):
    mb = m_ref[...]
    z = v_ref[...][:, None] + u_ref[...][None, :]
    raw = jnp.where(z >= 0, z, 0.2 * z)
    masked = jnp.where(mb > 0, raw, NEG)
    cmax = jnp.max(masked, axis=0, keepdims=True)
    cmax = jnp.where(cmax <= NEG * 0.5, 0.0, cmax)
    e = jnp.where(mb > 0, jnp.exp(raw - cmax), 0.0)
    ssum = jnp.sum(e, axis=0, keepdims=True)
    a_ref[...] = e / (ssum + 1e-16)
    cnt = jnp.sum(mb, axis=0)
    dn_ref[...] = jnp.where(cnt > 0, 1.0 / cnt, 0.0)


def col_softmax(M, u, v):
    return pl.pallas_call(
        _colsm_body,
        out_shape=[
            jax.ShapeDtypeStruct((BUF, BUF), jnp.float32),
            jax.ShapeDtypeStruct((BUF,), jnp.float32),
        ],
        grid=(BUF // _CB,),
        in_specs=[
            pl.BlockSpec((BUF, _CB), lambda j: (0, j)),
            pl.BlockSpec((_CB,), lambda j: (j,)),
            pl.BlockSpec((BUF,), lambda j: (0,)),
        ],
        out_specs=[
            pl.BlockSpec((BUF, _CB), lambda j: (0, j)),
            pl.BlockSpec((_CB,), lambda j: (j,)),
        ],
    )(M, u, v)


def _oute_body(a_ref, xl_ref, o_ref):
    o_ref[...] = jnp.dot(
        a_ref[...], xl_ref[...], preferred_element_type=jnp.float32
    ) * (1.0 / K)


def out_e_mm(A, Xl):
    return pl.pallas_call(
        _oute_body,
        out_shape=jax.ShapeDtypeStruct((BUF, FEAT), jnp.float32),
        grid=(BUF // _PB,),
        in_specs=[
            pl.BlockSpec((_PB, BUF), lambda i: (i, 0)),
            pl.BlockSpec((BUF, FEAT), lambda i: (0, 0)),
        ],
        out_specs=pl.BlockSpec((_PB, FEAT), lambda i: (i, 0)),
    )(A, Xl)


def _outn_body(a_ref, oe_ref, dn_ref, b_ref, o_ref):
    ob = jax.lax.dot_general(
        a_ref[...], oe_ref[...],
        dimension_numbers=(((0,), (0,)), ((), ())),
        preferred_element_type=jnp.float32,
    )
    o_ref[...] = ob * dn_ref[...][:, None] + b_ref[...][None, :]


def out_node_mm(A, out_e, dn, bias):
    return pl.pallas_call(
        _outn_body,
        out_shape=jax.ShapeDtypeStruct((BUF, FEAT), jnp.float32),
        grid=(BUF // _PB,),
        in_specs=[
            pl.BlockSpec((BUF, _PB), lambda j: (0, j)),
            pl.BlockSpec((BUF, FEAT), lambda j: (0, 0)),
            pl.BlockSpec((_PB,), lambda j: (j,)),
            pl.BlockSpec((FEAT,), lambda j: (0,)),
        ],
        out_specs=pl.BlockSpec((_PB, FEAT), lambda j: (j, 0)),
    )(A, out_e, dn, bias)


def _normfc_body(x_ref, gw_ref, gb_ref, gm_ref, wf_ref, bf_ref,
                 xn_ref, o1_ref):
    x = x_ref[...]
    mean = jnp.mean(x, axis=0, keepdims=True)
    cen = x - gm_ref[...][None, :] * mean
    var = jnp.mean(cen * cen, axis=0, keepdims=True)
    xn = cen / jnp.sqrt(var + 1e-5) * gw_ref[...][None, :] + gb_ref[...][None, :]
    xn = jnp.where(xn >= 0, xn, 0.01 * xn)
    xn_ref[...] = xn
    o1 = jnp.dot(xn, wf_ref[...], preferred_element_type=jnp.float32) + bf_ref[...][None, :]
    o1_ref[...] = jnp.where(o1 >= 0, o1, 0.01 * o1)


def norm_fc(x, gw, gb, gm, Wf, bf):
    hg = Wf.shape[1]
    return pl.pallas_call(
        _normfc_body,
        out_shape=[
            jax.ShapeDtypeStruct((BUF, FEAT), jnp.float32),
            jax.ShapeDtypeStruct((BUF, hg), jnp.float32),
        ],
        grid=(1,),
        in_specs=[
            pl.BlockSpec((BUF, FEAT), lambda i: (0, 0)),
            pl.BlockSpec((FEAT,), lambda i: (0,)),
            pl.BlockSpec((FEAT,), lambda i: (0,)),
            pl.BlockSpec((FEAT,), lambda i: (0,)),
            pl.BlockSpec((FEAT, hg), lambda i: (0, 0)),
            pl.BlockSpec((hg,), lambda i: (0,)),
        ],
        out_specs=[
            pl.BlockSpec((BUF, FEAT), lambda i: (0, 0)),
            pl.BlockSpec((BUF, hg), lambda i: (0, 0)),
        ],
    )(x, gw, gb, gm, Wf, bf)


_AB = 512


def _att_body(out_ref, w1_ref, b1_ref, w2_ref, b2_ref, s_ref):
    j = pl.program_id(0)
    T = jax.lax.dot_general(
        out_ref[...], w1_ref[...],
        dimension_numbers=(((0,), (0,)), ((), ())),
        preferred_element_type=jnp.float32,
    )
    T = jax.nn.relu(T + b1_ref[...][None, :])
    sp = jnp.sum(T * w2_ref[...][None, :], axis=1)

    @pl.when(j == 0)
    def _():
        s_ref[...] = sp + b2_ref[...]

    @pl.when(j != 0)
    def _():
        s_ref[...] += sp


def att_scores(out, Wat1, bat1, Wat2, bat2):
    D = out.shape[1]
    b2 = jnp.broadcast_to(bat2, (D,))
    return pl.pallas_call(
        _att_body,
        out_shape=jax.ShapeDtypeStruct((D,), jnp.float32),
        grid=(BUF // _AB,),
        in_specs=[
            pl.BlockSpec((BUF, D), lambda j: (0, 0)),
            pl.BlockSpec((BUF, _AB), lambda j: (0, j)),
            pl.BlockSpec((_AB,), lambda j: (j,)),
            pl.BlockSpec((_AB,), lambda j: (j,)),
            pl.BlockSpec((D,), lambda j: (0,)),
        ],
        out_specs=pl.BlockSpec((D,), lambda j: (0,)),
    )(out, Wat1, bat1, Wat2[:, 0], b2)


def _heads_body(s_ref, o_ref, wc1_ref, bc1_ref, wd_ref, bd_ref,
                wc2_ref, bc2_ref, wch_ref, bch_ref, d_ref, lg_ref):
    s = jax.nn.sigmoid(s_ref[...])
    s = s - jnp.mean(s)
    H = o_ref[...] * s[None, :]
    H1 = jnp.dot(H, wc1_ref[...], preferred_element_type=jnp.float32) + bc1_ref[...][None, :]
    H1 = jnp.where(H1 >= 0, H1, 0.01 * H1)
    d_ref[...] = jnp.dot(H1, wd_ref[...], preferred_element_type=jnp.float32) + bd_ref[...][None, :]
    H2 = jnp.dot(H1, wc2_ref[...], preferred_element_type=jnp.float32) + bc2_ref[...][None, :] + H1
    H2 = jnp.where(H2 >= 0, H2, 0.01 * H2)
    lg_ref[...] = jnp.dot(H2, wch_ref[...], preferred_element_type=jnp.float32) + bch_ref[...][None, :]


def heads(s, out20, g):
    hg = g['Wc2'].shape[0]
    D = out20.shape[1]
    o_pad = jnp.pad(out20, ((0, 32 - out20.shape[0]), (0, 0)))
    wd = jnp.pad(g['Wd'], ((0, 0), (0, 128 - NCLS)))
    bd = jnp.pad(g['bd'], (0, 128 - NCLS))
    wch = jnp.pad(g['Wch'], ((0, 0), (0, 128 - NCLS)))
    bch = jnp.pad(g['bch'], (0, 128 - NCLS))
    full = lambda shape: pl.BlockSpec(shape, lambda: tuple(0 for _ in shape))
    d, lg = pl.pallas_call(
        _heads_body,
        out_shape=[
            jax.ShapeDtypeStruct((32, 128), jnp.float32),
            jax.ShapeDtypeStruct((32, 128), jnp.float32),
        ],
        in_specs=[
            full((D,)), full((32, D)), full((D, hg)), full((hg,)),
            full((hg, 128)), full((128,)), full((hg, hg)), full((hg,)),
            full((hg, 128)), full((128,)),
        ],
        out_specs=[full((32, 128)), full((32, 128))],
    )(s, o_pad, g['Wc1'], g['bc1'], wd, bd, g['Wc2'], g['bc2'], wch, bch)
    return d[:BATCH, :NCLS], lg[:BATCH, :NCLS]


def layer_norm(x, g, b, eps=1e-5):
    m = x.mean(-1, keepdims=True)
    v = ((x - m) ** 2).mean(-1, keepdims=True)
    return (x - m) / jnp.sqrt(v + eps) * g + b


def seg_softmax(vals, seg, num):
    m = jax.ops.segment_max(vals, seg, num)
    m = jnp.where(jnp.isneginf(m), 0.0, m)
    e = jnp.exp(vals - m[seg])
    s = jax.ops.segment_sum(e, seg, num)
    return e / (s[seg] + 1e-16)


def genconv(x, src, dst, c):
    msg = jax.nn.relu(x[src]) + 1e-7
    alpha = seg_softmax(msg * c['t'], dst, x.shape[0])
    out = jax.ops.segment_sum(msg * alpha, dst, x.shape[0])
    out = out + x
    h = out @ c['W1'] + c['b1']
    h = layer_norm(h, c['g1'], c['bt1'])
    h = jax.nn.relu(h)
    return h @ c['W2'] + c['b2']


def hypergraph_block(xc, params):
    g = params['gcn']
    M = topk_mask(xc)
    Xl1, u1, v1 = hyper_proj(xc, g['Whg1'], g['att1'])
    A1, dn1 = col_softmax(M, u1, v1)
    oute1 = out_e_mm(A1, Xl1)
    nx1 = out_node_mm(A1, oute1, dn1, g['bhg1'])
    xn1, out1 = norm_fc(nx1, g['gw1'], g['gb1'], g['gm1'], g['Wfc1'], g['bfc1'])
    Xl2, u2, v2 = hyper_proj(xn1, g['Whg2'], g['att2'], ea=xc)
    A2, dn2 = col_softmax(M, u2, v2)
    oute2 = out_e_mm(A2, Xl2)
    nx2 = out_node_mm(A2, oute2, dn2, g['bhg2'])
    xn2, out2 = norm_fc(nx2, g['gw2'], g['gb2'], g['gm2'], g['Wfc2'], g['bfc2'])

    out = jnp.concatenate([xc, out1, out2], axis=1)
    s = att_scores(out, g['Wat1'], g['bat1'], g['Wat2'], g['bat2'])
    return heads(s, out[:BATCH], g)


def kernel(x, edge_index, edge_latent, y, params):
    p = params
    src, dst = edge_index[0], edge_index[1]
    x = fc_relu(x, p['W_fc'], p['b_fc'])
    x_ = x
    x = genconv(x_, src, dst, p['convs'][0])
    x_ = jnp.concatenate([x_, x], axis=-1)
    for i in (1, 2):
        h = genconv(x, src, dst, p['convs'][i])
        h = layer_norm(h, p['lns'][i - 1]['g'], p['lns'][i - 1]['b'])
        h = jax.nn.relu(h)
        x = x + h
        x_ = jnp.concatenate([x_, x], axis=-1)
    h_path = x_.reshape(BATCH, 500, 4 * HID)
    h_path = jax.nn.relu(h_path @ p['Wphi'] + p['bphi'])
    a = jnp.tanh(h_path @ p['Wa'] + p['ba'])
    bgate = jax.nn.sigmoid(h_path @ p['Wb'] + p['bb'])
    A = (a * bgate) @ p['Wc'] + p['bc']
    A = jnp.swapaxes(A, -1, -2)
    h_path = jax.nn.softmax(A, axis=-1) @ h_path
    h = jax.nn.relu(h_path @ p['Wrho'] + p['brho'])[:, 0, :]
    logits = h @ p['Wcls'] + p['bcls']
    x_concat = jnp.concatenate([h, p['rehearsal']], axis=0)[:BUF]
    d, lg = hypergraph_block(x_concat, p)
    return logits, lg, d


# speedup vs baseline: 2.2371x; 1.0004x over previous
import functools

import jax
import jax.numpy as jnp
from jax import lax
from jax.experimental import pallas as pl
from jax.experimental.pallas import tpu as pltpu
from jax.experimental.pallas import tpu_sc as plsc

N_NODES = 10000
N_EDGES = 160000
FEAT = 512
HID = 128
BUF = 4096
K = 16
BATCH = 20
NCLS = 4
NEG = -3.0e38


def _fc_body(x_ref, w_ref, b_ref, o_ref):
    o_ref[...] = jax.nn.relu(
        jnp.dot(x_ref[...], w_ref[...], preferred_element_type=jnp.float32)
        + b_ref[...]
    )


def fc_relu(x, w, b):
    n = x.shape[0]
    pad = (-n) % 8
    xp = jnp.pad(x, ((0, pad), (0, 0)))
    out = pl.pallas_call(
        _fc_body,
        out_shape=jax.ShapeDtypeStruct((n + pad, w.shape[1]), jnp.float32),
        grid=(1,),
        in_specs=[
            pl.BlockSpec((n + pad, x.shape[1]), lambda i: (0, 0)),
            pl.BlockSpec((w.shape[0], w.shape[1]), lambda i: (0, 0)),
            pl.BlockSpec((w.shape[1],), lambda i: (0,)),
        ],
        out_specs=pl.BlockSpec((n + pad, w.shape[1]), lambda i: (0, 0)),
    )(xp, w, b)
    return out[:n]


_TKR = 128


def _topk_body(xb_ref, xcT_ref, m_ref, sim_ref):
    sim_ref[...] = jnp.dot(
        xb_ref[...], xcT_ref[...], preferred_element_type=jnp.float32
    )
    m_ref[...] = jnp.zeros_like(m_ref)
    cols = jax.lax.broadcasted_iota(jnp.int32, (_TKR, BUF), 1)

    def step(t, carry):
        s = sim_ref[...]
        rmax = jnp.max(s, axis=1, keepdims=True)
        pick = jnp.min(
            jnp.where(s == rmax, cols, BUF), axis=1, keepdims=True
        )
        hit = cols == pick
        m_ref[...] += hit.astype(jnp.float32)
        sim_ref[...] = jnp.where(hit, NEG, s)
        return carry

    jax.lax.fori_loop(0, K, step, 0)


def topk_mask(xc):
    xcT = xc.T
    return pl.pallas_call(
        _topk_body,
        out_shape=jax.ShapeDtypeStruct((BUF, BUF), jnp.float32),
        grid=(BUF // _TKR,),
        in_specs=[
            pl.BlockSpec((_TKR, FEAT), lambda i: (i, 0)),
            pl.BlockSpec((FEAT, BUF), lambda i: (0, 0)),
        ],
        out_specs=pl.BlockSpec((_TKR, BUF), lambda i: (i, 0)),
        scratch_shapes=[pltpu.VMEM((_TKR, BUF), jnp.float32)],
    )(xc, xcT)


def _proj_he_body(x_ref, ea_ref, w_ref, aA_ref, aB_ref, xl_ref, u_ref, v_ref):
    xl = jnp.dot(x_ref[...], w_ref[...], preferred_element_type=jnp.float32)
    xl_ref[...] = xl
    u_ref[...] = jnp.sum(xl * aA_ref[...][None, :], axis=1)
    he = jnp.dot(ea_ref[...], w_ref[...], preferred_element_type=jnp.float32)
    v_ref[...] = jnp.sum(he * aB_ref[...][None, :], axis=1)


def _proj_same_body(x_ref, w_ref, aA_ref, aB_ref, xl_ref, u_ref, v_ref):
    xl = jnp.dot(x_ref[...], w_ref[...], preferred_element_type=jnp.float32)
    xl_ref[...] = xl
    u_ref[...] = jnp.sum(xl * aA_ref[...][None, :], axis=1)
    v_ref[...] = jnp.sum(xl * aB_ref[...][None, :], axis=1)


_PB = 512


def hyper_proj(x, W, att, ea=None):
    attA, attB = att[:FEAT], att[FEAT:]
    out_shape = [
        jax.ShapeDtypeStruct((BUF, FEAT), jnp.float32),
        jax.ShapeDtypeStruct((BUF,), jnp.float32),
        jax.ShapeDtypeStruct((BUF,), jnp.float32),
    ]
    out_specs = [
        pl.BlockSpec((_PB, FEAT), lambda i: (i, 0)),
        pl.BlockSpec((_PB,), lambda i: (i,)),
        pl.BlockSpec((_PB,), lambda i: (i,)),
    ]
    wspec = pl.BlockSpec((FEAT, FEAT), lambda i: (0, 0))
    aspec = pl.BlockSpec((FEAT,), lambda i: (0,))
    xspec = pl.BlockSpec((_PB, FEAT), lambda i: (i, 0))
    if ea is None:
        return pl.pallas_call(
            _proj_same_body,
            out_shape=out_shape,
            grid=(BUF // _PB,),
            in_specs=[xspec, wspec, aspec, aspec],
            out_specs=out_specs,
        )(x, W, attA, attB)
    return pl.pallas_call(
        _proj_he_body,
        out_shape=out_shape,
        grid=(BUF // _PB,),
        in_specs=[xspec, xspec, wspec, aspec, aspec],
        out_specs=out_specs,
    )(x, ea, W, attA, attB)


_CB = 512


def _colsm_body(m_ref, u_ref, v_ref, a_ref, dn_ref):
    mb = m_ref[...]
    z = v_ref[...][:, None] + u_ref[...][None, :]
    raw = jnp.where(z >= 0, z, 0.2 * z)
    masked = jnp.where(mb > 0, raw, NEG)
    cmax = jnp.max(masked, axis=0, keepdims=True)
    cmax = jnp.where(cmax <= NEG * 0.5, 0.0, cmax)
    e = jnp.where(mb > 0, jnp.exp(raw - cmax), 0.0)
    ssum = jnp.sum(e, axis=0, keepdims=True)
    a_ref[...] = e / (ssum + 1e-16)
    cnt = jnp.sum(mb, axis=0)
    dn_ref[...] = jnp.where(cnt > 0, 1.0 / cnt, 0.0)


def col_softmax(M, u, v):
    return pl.pallas_call(
        _colsm_body,
        out_shape=[
            jax.ShapeDtypeStruct((BUF, BUF), jnp.float32),
            jax.ShapeDtypeStruct((BUF,), jnp.float32),
        ],
        grid=(BUF // _CB,),
        in_specs=[
            pl.BlockSpec((BUF, _CB), lambda j: (0, j)),
            pl.BlockSpec((_CB,), lambda j: (j,)),
            pl.BlockSpec((BUF,), lambda j: (0,)),
        ],
        out_specs=[
            pl.BlockSpec((BUF, _CB), lambda j: (0, j)),
            pl.BlockSpec((_CB,), lambda j: (j,)),
        ],
    )(M, u, v)


def _oute_body(a_ref, xl_ref, o_ref):
    o_ref[...] = jnp.dot(
        a_ref[...], xl_ref[...], preferred_element_type=jnp.float32
    ) * (1.0 / K)


def out_e_mm(A, Xl):
    return pl.pallas_call(
        _oute_body,
        out_shape=jax.ShapeDtypeStruct((BUF, FEAT), jnp.float32),
        grid=(BUF // _PB,),
        in_specs=[
            pl.BlockSpec((_PB, BUF), lambda i: (i, 0)),
            pl.BlockSpec((BUF, FEAT), lambda i: (0, 0)),
        ],
        out_specs=pl.BlockSpec((_PB, FEAT), lambda i: (i, 0)),
    )(A, Xl)


def _outn_body(a_ref, oe_ref, dn_ref, b_ref, o_ref):
    ob = jax.lax.dot_general(
        a_ref[...], oe_ref[...],
        dimension_numbers=(((0,), (0,)), ((), ())),
        preferred_element_type=jnp.float32,
    )
    o_ref[...] = ob * dn_ref[...][:, None] + b_ref[...][None, :]


def out_node_mm(A, out_e, dn, bias):
    return pl.pallas_call(
        _outn_body,
        out_shape=jax.ShapeDtypeStruct((BUF, FEAT), jnp.float32),
        grid=(BUF // _PB,),
        in_specs=[
            pl.BlockSpec((BUF, _PB), lambda j: (0, j)),
            pl.BlockSpec((BUF, FEAT), lambda j: (0, 0)),
            pl.BlockSpec((_PB,), lambda j: (j,)),
            pl.BlockSpec((FEAT,), lambda j: (0,)),
        ],
        out_specs=pl.BlockSpec((_PB, FEAT), lambda j: (j, 0)),
    )(A, out_e, dn, bias)


def _normfc_body(x_ref, gw_ref, gb_ref, gm_ref, wf_ref, bf_ref,
                 xn_ref, o1_ref):
    x = x_ref[...]
    mean = jnp.mean(x, axis=0, keepdims=True)
    cen = x - gm_ref[...][None, :] * mean
    var = jnp.mean(cen * cen, axis=0, keepdims=True)
    xn = cen / jnp.sqrt(var + 1e-5) * gw_ref[...][None, :] + gb_ref[...][None, :]
    xn = jnp.where(xn >= 0, xn, 0.01 * xn)
    xn_ref[...] = xn
    o1 = jnp.dot(xn, wf_ref[...], preferred_element_type=jnp.float32) + bf_ref[...][None, :]
    o1_ref[...] = jnp.where(o1 >= 0, o1, 0.01 * o1)


def norm_fc(x, gw, gb, gm, Wf, bf):
    hg = Wf.shape[1]
    return pl.pallas_call(
        _normfc_body,
        out_shape=[
            jax.ShapeDtypeStruct((BUF, FEAT), jnp.float32),
            jax.ShapeDtypeStruct((BUF, hg), jnp.float32),
        ],
        grid=(1,),
        in_specs=[
            pl.BlockSpec((BUF, FEAT), lambda i: (0, 0)),
            pl.BlockSpec((FEAT,), lambda i: (0,)),
            pl.BlockSpec((FEAT,), lambda i: (0,)),
            pl.BlockSpec((FEAT,), lambda i: (0,)),
            pl.BlockSpec((FEAT, hg), lambda i: (0, 0)),
            pl.BlockSpec((hg,), lambda i: (0,)),
        ],
        out_specs=[
            pl.BlockSpec((BUF, FEAT), lambda i: (0, 0)),
            pl.BlockSpec((BUF, hg), lambda i: (0, 0)),
        ],
    )(x, gw, gb, gm, Wf, bf)


_AB = 512


def _att_body(out_ref, w1_ref, b1_ref, w2_ref, b2_ref, s_ref):
    j = pl.program_id(0)
    T = jax.lax.dot_general(
        out_ref[...], w1_ref[...],
        dimension_numbers=(((0,), (0,)), ((), ())),
        preferred_element_type=jnp.float32,
    )
    T = jax.nn.relu(T + b1_ref[...][None, :])
    sp = jnp.sum(T * w2_ref[...][None, :], axis=1)

    @pl.when(j == 0)
    def _():
        s_ref[...] = sp + b2_ref[...]

    @pl.when(j != 0)
    def _():
        s_ref[...] += sp


def att_scores(out, Wat1, bat1, Wat2, bat2):
    D = out.shape[1]
    b2 = jnp.broadcast_to(bat2, (D,))
    return pl.pallas_call(
        _att_body,
        out_shape=jax.ShapeDtypeStruct((D,), jnp.float32),
        grid=(BUF // _AB,),
        in_specs=[
            pl.BlockSpec((BUF, D), lambda j: (0, 0)),
            pl.BlockSpec((BUF, _AB), lambda j: (0, j)),
            pl.BlockSpec((_AB,), lambda j: (j,)),
            pl.BlockSpec((_AB,), lambda j: (j,)),
            pl.BlockSpec((D,), lambda j: (0,)),
        ],
        out_specs=pl.BlockSpec((D,), lambda j: (0,)),
    )(out, Wat1, bat1, Wat2[:, 0], b2)


def _heads_body(s_ref, o_ref, wc1_ref, bc1_ref, wd_ref, bd_ref,
                wc2_ref, bc2_ref, wch_ref, bch_ref, d_ref, lg_ref):
    s = jax.nn.sigmoid(s_ref[...])
    s = s - jnp.mean(s)
    H = o_ref[...] * s[None, :]
    H1 = jnp.dot(H, wc1_ref[...], preferred_element_type=jnp.float32) + bc1_ref[...][None, :]
    H1 = jnp.where(H1 >= 0, H1, 0.01 * H1)
    d_ref[...] = jnp.dot(H1, wd_ref[...], preferred_element_type=jnp.float32) + bd_ref[...][None, :]
    H2 = jnp.dot(H1, wc2_ref[...], preferred_element_type=jnp.float32) + bc2_ref[...][None, :] + H1
    H2 = jnp.where(H2 >= 0, H2, 0.01 * H2)
    lg_ref[...] = jnp.dot(H2, wch_ref[...], preferred_element_type=jnp.float32) + bch_ref[...][None, :]


def heads(s, out20, g):
    hg = g['Wc2'].shape[0]
    D = out20.shape[1]
    o_pad = jnp.pad(out20, ((0, 32 - out20.shape[0]), (0, 0)))
    wd = jnp.pad(g['Wd'], ((0, 0), (0, 128 - NCLS)))
    bd = jnp.pad(g['bd'], (0, 128 - NCLS))
    wch = jnp.pad(g['Wch'], ((0, 0), (0, 128 - NCLS)))
    bch = jnp.pad(g['bch'], (0, 128 - NCLS))
    full = lambda shape: pl.BlockSpec(shape, lambda: tuple(0 for _ in shape))
    d, lg = pl.pallas_call(
        _heads_body,
        out_shape=[
            jax.ShapeDtypeStruct((32, 128), jnp.float32),
            jax.ShapeDtypeStruct((32, 128), jnp.float32),
        ],
        in_specs=[
            full((D,)), full((32, D)), full((D, hg)), full((hg,)),
            full((hg, 128)), full((128,)), full((hg, hg)), full((hg,)),
            full((hg, 128)), full((128,)),
        ],
        out_specs=[full((32, 128)), full((32, 128))],
    )(s, o_pad, g['Wc1'], g['bc1'], wd, bd, g['Wc2'], g['bc2'], wch, bch)
    return d[:BATCH, :NCLS], lg[:BATCH, :NCLS]


_EC = 125
_TILES = 16
_EPT = N_EDGES // _TILES
_NCH = _EPT // _EC
_NPAD = 10240
_RPT = _NPAD // _TILES
_HF = HID // 2


def _gen_sc_body(xflat, srcr, dstr, tvec_h, zrows, out_h,
                 srcv, dstv, gbuf0, gbuf1, stg0, stg1, tv,
                 acc, gsem0, gsem1, ssem0, ssem1):
    cid = lax.axis_index("c")
    sid = lax.axis_index("s")
    pltpu.sync_copy(srcr.at[pl.ds(cid * (N_EDGES // _EC) + sid * _NCH, _NCH)],
                    srcv)
    pltpu.sync_copy(dstr.at[pl.ds(sid * _NCH, _NCH)], dstv)
    pltpu.sync_copy(tvec_h, tv)
    pltpu.sync_copy(zrows, acc.at[pl.ds(sid * _RPT, _RPT)])
    plsc.subcore_barrier()
    t16 = tv[...]

    def compute(gbuf, stg):
        def row(r, carry):
            for c in range(_HF // 16):
                v = gbuf[r, pl.ds(c * 16, 16)]
                m = jnp.maximum(v, 0.0) + 1e-7
                e = jnp.exp(m * t16)
                stg[r, pl.ds(c * 16, 16)] = e
                stg[r, pl.ds(_HF + c * 16, 16)] = m * e
            return carry
        lax.fori_loop(0, _EC, row, 0, unroll=2)

    def chunk(k, carry):
        pltpu.async_copy(xflat.at[srcv.at[k]], gbuf0, gsem0).wait()
        compute(gbuf0, stg0)
        pltpu.async_copy(stg0, acc.at[dstv.at[k]], ssem0, add=True).wait()
        return carry

    lax.fori_loop(0, _NCH, chunk, 0)
    plsc.subcore_barrier()
    pltpu.sync_copy(acc.at[pl.ds(sid * _RPT, _RPT)],
                    out_h.at[pl.ds(cid * _NPAD + sid * _RPT, _RPT)])


def genconv_sc(x, src_off, dst_r, t):
    xflat = jnp.concatenate([x[:, :_HF], x[:, _HF:]], axis=0)
    tvec = jnp.broadcast_to(t.astype(jnp.float32).reshape(()), (16,))
    zrows = jnp.zeros((_RPT, HID), jnp.float32)
    mesh = plsc.VectorSubcoreMesh(core_axis_name="c", subcore_axis_name="s")
    acc_out = pl.kernel(
        _gen_sc_body,
        out_type=jax.ShapeDtypeStruct((2 * _NPAD, HID), jnp.float32),
        mesh=mesh,
        scratch_types=[
            pltpu.VMEM((_NCH, _EC), jnp.int32),
            pltpu.VMEM((_NCH, _EC), jnp.int32),
            pltpu.VMEM((_EC, _HF), jnp.float32),
            pltpu.VMEM((_EC, _HF), jnp.float32),
            pltpu.VMEM((_EC, HID), jnp.float32),
            pltpu.VMEM((_EC, HID), jnp.float32),
            pltpu.VMEM((16,), jnp.float32),
            pltpu.VMEM_SHARED((_NPAD, HID), jnp.float32),
            pltpu.SemaphoreType.DMA,
            pltpu.SemaphoreType.DMA,
            pltpu.SemaphoreType.DMA,
            pltpu.SemaphoreType.DMA,
        ],
        compiler_params=pltpu.CompilerParams(use_tc_tiling_on_sc=False),
    )(xflat, src_off, dst_r, tvec, zrows)
    den = jnp.concatenate([acc_out[:N_NODES, :_HF],
                           acc_out[_NPAD:_NPAD + N_NODES, :_HF]], axis=1)
    num = jnp.concatenate([acc_out[:N_NODES, _HF:],
                           acc_out[_NPAD:_NPAD + N_NODES, _HF:]], axis=1)
    return num, den


def _gcmlp_body(x_ref, num_ref, den_ref, w1_ref, b1_ref, g1_ref, bt1_ref,
                w2_ref, b2_ref, lg_ref, lb_ref, o_ref, *, post):
    x = x_ref[...]
    agg = num_ref[...] / (den_ref[...] + 1e-16) + x
    h = jnp.dot(agg, w1_ref[...], preferred_element_type=jnp.float32) + b1_ref[...][None, :]
    mu = jnp.mean(h, axis=1, keepdims=True)
    var = jnp.mean((h - mu) ** 2, axis=1, keepdims=True)
    h = (h - mu) / jnp.sqrt(var + 1e-5) * g1_ref[...][None, :] + bt1_ref[...][None, :]
    h = jax.nn.relu(h)
    o = jnp.dot(h, w2_ref[...], preferred_element_type=jnp.float32) + b2_ref[...][None, :]
    if post:
        mu2 = jnp.mean(o, axis=1, keepdims=True)
        var2 = jnp.mean((o - mu2) ** 2, axis=1, keepdims=True)
        o = (o - mu2) / jnp.sqrt(var2 + 1e-5) * lg_ref[...][None, :] + lb_ref[...][None, :]
        o = jax.nn.relu(o)
        o = x + o
    o_ref[...] = o


_GB = 1000


def gc_mlp(x, num, den, c, ln=None):
    post = ln is not None
    lg = ln['g'] if post else c['b2']
    lb = ln['b'] if post else c['b2']
    h2 = c['W1'].shape[1]
    body = functools.partial(_gcmlp_body, post=post)
    vec = lambda n: pl.BlockSpec((n,), lambda i: (0,))
    return pl.pallas_call(
        body,
        out_shape=jax.ShapeDtypeStruct((N_NODES, HID), jnp.float32),
        grid=(N_NODES // _GB,),
        in_specs=[
            pl.BlockSpec((_GB, HID), lambda i: (i, 0)),
            pl.BlockSpec((_GB, HID), lambda i: (i, 0)),
            pl.BlockSpec((_GB, HID), lambda i: (i, 0)),
            pl.BlockSpec((HID, h2), lambda i: (0, 0)),
            vec(h2),
            vec(h2),
            vec(h2),
            pl.BlockSpec((h2, HID), lambda i: (0, 0)),
            vec(HID),
            vec(HID),
            vec(HID),
        ],
        out_specs=pl.BlockSpec((_GB, HID), lambda i: (i, 0)),
    )(x, num, den, c['W1'], c['b1'], c['g1'], c['bt1'], c['W2'], c['b2'],
      lg, lb)


def genconv_fused(x, src_off, dst_r, c, ln=None):
    num, den = genconv_sc(x, src_off, dst_r, c['t'])
    return gc_mlp(x, num, den, c, ln)


def layer_norm(x, g, b, eps=1e-5):
    m = x.mean(-1, keepdims=True)
    v = ((x - m) ** 2).mean(-1, keepdims=True)
    return (x - m) / jnp.sqrt(v + eps) * g + b


def seg_softmax(vals, seg, num):
    m = jax.ops.segment_max(vals, seg, num)
    m = jnp.where(jnp.isneginf(m), 0.0, m)
    e = jnp.exp(vals - m[seg])
    s = jax.ops.segment_sum(e, seg, num)
    return e / (s[seg] + 1e-16)


def genconv(x, src, dst, c):
    msg = jax.nn.relu(x[src]) + 1e-7
    alpha = seg_softmax(msg * c['t'], dst, x.shape[0])
    out = jax.ops.segment_sum(msg * alpha, dst, x.shape[0])
    out = out + x
    h = out @ c['W1'] + c['b1']
    h = layer_norm(h, c['g1'], c['bt1'])
    h = jax.nn.relu(h)
    return h @ c['W2'] + c['b2']


def hypergraph_block(xc, params):
    g = params['gcn']
    M = topk_mask(xc)
    Xl1, u1, v1 = hyper_proj(xc, g['Whg1'], g['att1'])
    A1, dn1 = col_softmax(M, u1, v1)
    oute1 = out_e_mm(A1, Xl1)
    nx1 = out_node_mm(A1, oute1, dn1, g['bhg1'])
    xn1, out1 = norm_fc(nx1, g['gw1'], g['gb1'], g['gm1'], g['Wfc1'], g['bfc1'])
    Xl2, u2, v2 = hyper_proj(xn1, g['Whg2'], g['att2'], ea=xc)
    A2, dn2 = col_softmax(M, u2, v2)
    oute2 = out_e_mm(A2, Xl2)
    nx2 = out_node_mm(A2, oute2, dn2, g['bhg2'])
    xn2, out2 = norm_fc(nx2, g['gw2'], g['gb2'], g['gm2'], g['Wfc2'], g['bfc2'])

    out = jnp.concatenate([xc, out1, out2], axis=1)
    s = att_scores(out, g['Wat1'], g['bat1'], g['Wat2'], g['bat2'])
    return heads(s, out[:BATCH], g)


def kernel(x, edge_index, edge_latent, y, params):
    p = params
    src, dst = edge_index[0], edge_index[1]
    x = fc_relu(x, p['W_fc'], p['b_fc'])
    x_ = x
    x = genconv(x_, src, dst, p['convs'][0])
    x_ = jnp.concatenate([x_, x], axis=-1)
    for i in (1, 2):
        h = genconv(x, src, dst, p['convs'][i])
        h = layer_norm(h, p['lns'][i - 1]['g'], p['lns'][i - 1]['b'])
        h = jax.nn.relu(h)
        x = x + h
        x_ = jnp.concatenate([x_, x], axis=-1)
    h_path = x_.reshape(BATCH, 500, 4 * HID)
    h_path = jax.nn.relu(h_path @ p['Wphi'] + p['bphi'])
    a = jnp.tanh(h_path @ p['Wa'] + p['ba'])
    bgate = jax.nn.sigmoid(h_path @ p['Wb'] + p['bb'])
    A = (a * bgate) @ p['Wc'] + p['bc']
    A = jnp.swapaxes(A, -1, -2)
    h_path = jax.nn.softmax(A, axis=-1) @ h_path
    h = jax.nn.relu(h_path @ p['Wrho'] + p['brho'])[:, 0, :]
    logits = h @ p['Wcls'] + p['bcls']
    x_concat = jnp.concatenate([h, p['rehearsal']], axis=0)[:BUF]
    d, lg = hypergraph_block(x_concat, p)
    return logits, lg, d


# speedup vs baseline: 2.2726x; 1.0159x over previous
import functools

import jax
import jax.numpy as jnp
from jax import lax
from jax.experimental import pallas as pl
from jax.experimental.pallas import tpu as pltpu
from jax.experimental.pallas import tpu_sc as plsc

N_NODES = 10000
N_EDGES = 160000
FEAT = 512
HID = 128
BUF = 4096
K = 16
BATCH = 20
NCLS = 4
NEG = -3.0e38


def _fc_body(x_ref, w_ref, b_ref, o_ref):
    o_ref[...] = jax.nn.relu(
        jnp.dot(x_ref[...], w_ref[...], preferred_element_type=jnp.float32)
        + b_ref[...]
    )


def fc_relu(x, w, b):
    n = x.shape[0]
    pad = (-n) % 8
    xp = jnp.pad(x, ((0, pad), (0, 0)))
    out = pl.pallas_call(
        _fc_body,
        out_shape=jax.ShapeDtypeStruct((n + pad, w.shape[1]), jnp.float32),
        grid=(1,),
        in_specs=[
            pl.BlockSpec((n + pad, x.shape[1]), lambda i: (0, 0)),
            pl.BlockSpec((w.shape[0], w.shape[1]), lambda i: (0, 0)),
            pl.BlockSpec((w.shape[1],), lambda i: (0,)),
        ],
        out_specs=pl.BlockSpec((n + pad, w.shape[1]), lambda i: (0, 0)),
    )(xp, w, b)
    return out[:n]


_TKR = 128


def _topk_body(xb_ref, xcT_ref, m_ref, sim_ref):
    sim_ref[...] = jnp.dot(
        xb_ref[...], xcT_ref[...], preferred_element_type=jnp.float32
    )
    m_ref[...] = jnp.zeros_like(m_ref)
    cols = jax.lax.broadcasted_iota(jnp.int32, (_TKR, BUF), 1)

    for t in range(K):
        s = sim_ref[...]
        rmax = jnp.max(s, axis=1, keepdims=True)
        pick = jnp.min(
            jnp.where(s == rmax, cols, BUF), axis=1, keepdims=True
        )
        hit = cols == pick
        m_ref[...] += hit.astype(jnp.float32)
        sim_ref[...] = jnp.where(hit, NEG, s)


def topk_mask(xc):
    xcT = xc.T
    return pl.pallas_call(
        _topk_body,
        out_shape=jax.ShapeDtypeStruct((BUF, BUF), jnp.float32),
        grid=(BUF // _TKR,),
        in_specs=[
            pl.BlockSpec((_TKR, FEAT), lambda i: (i, 0)),
            pl.BlockSpec((FEAT, BUF), lambda i: (0, 0)),
        ],
        out_specs=pl.BlockSpec((_TKR, BUF), lambda i: (i, 0)),
        scratch_shapes=[pltpu.VMEM((_TKR, BUF), jnp.float32)],
    )(xc, xcT)


def _proj_he_body(x_ref, ea_ref, w_ref, aA_ref, aB_ref, xl_ref, u_ref, v_ref):
    xl = jnp.dot(x_ref[...], w_ref[...], preferred_element_type=jnp.float32)
    xl_ref[...] = xl
    u_ref[...] = jnp.sum(xl * aA_ref[...][None, :], axis=1)
    he = jnp.dot(ea_ref[...], w_ref[...], preferred_element_type=jnp.float32)
    v_ref[...] = jnp.sum(he * aB_ref[...][None, :], axis=1)


def _proj_same_body(x_ref, w_ref, aA_ref, aB_ref, xl_ref, u_ref, v_ref):
    xl = jnp.dot(x_ref[...], w_ref[...], preferred_element_type=jnp.float32)
    xl_ref[...] = xl
    u_ref[...] = jnp.sum(xl * aA_ref[...][None, :], axis=1)
    v_ref[...] = jnp.sum(xl * aB_ref[...][None, :], axis=1)


_PB = 512


def hyper_proj(x, W, att, ea=None):
    attA, attB = att[:FEAT], att[FEAT:]
    out_shape = [
        jax.ShapeDtypeStruct((BUF, FEAT), jnp.float32),
        jax.ShapeDtypeStruct((BUF,), jnp.float32),
        jax.ShapeDtypeStruct((BUF,), jnp.float32),
    ]
    out_specs = [
        pl.BlockSpec((_PB, FEAT), lambda i: (i, 0)),
        pl.BlockSpec((_PB,), lambda i: (i,)),
        pl.BlockSpec((_PB,), lambda i: (i,)),
    ]
    wspec = pl.BlockSpec((FEAT, FEAT), lambda i: (0, 0))
    aspec = pl.BlockSpec((FEAT,), lambda i: (0,))
    xspec = pl.BlockSpec((_PB, FEAT), lambda i: (i, 0))
    if ea is None:
        return pl.pallas_call(
            _proj_same_body,
            out_shape=out_shape,
            grid=(BUF // _PB,),
            in_specs=[xspec, wspec, aspec, aspec],
            out_specs=out_specs,
        )(x, W, attA, attB)
    return pl.pallas_call(
        _proj_he_body,
        out_shape=out_shape,
        grid=(BUF // _PB,),
        in_specs=[xspec, xspec, wspec, aspec, aspec],
        out_specs=out_specs,
    )(x, ea, W, attA, attB)


_CB = 512


def _colsm_body(m_ref, u_ref, v_ref, a_ref, dn_ref):
    mb = m_ref[...]
    z = v_ref[...][:, None] + u_ref[...][None, :]
    raw = jnp.where(z >= 0, z, 0.2 * z)
    masked = jnp.where(mb > 0, raw, NEG)
    cmax = jnp.max(masked, axis=0, keepdims=True)
    cmax = jnp.where(cmax <= NEG * 0.5, 0.0, cmax)
    e = jnp.where(mb > 0, jnp.exp(raw - cmax), 0.0)
    ssum = jnp.sum(e, axis=0, keepdims=True)
    a_ref[...] = e / (ssum + 1e-16)
    cnt = jnp.sum(mb, axis=0)
    dn_ref[...] = jnp.where(cnt > 0, 1.0 / cnt, 0.0)


def col_softmax(M, u, v):
    return pl.pallas_call(
        _colsm_body,
        out_shape=[
            jax.ShapeDtypeStruct((BUF, BUF), jnp.float32),
            jax.ShapeDtypeStruct((BUF,), jnp.float32),
        ],
        grid=(BUF // _CB,),
        in_specs=[
            pl.BlockSpec((BUF, _CB), lambda j: (0, j)),
            pl.BlockSpec((_CB,), lambda j: (j,)),
            pl.BlockSpec((BUF,), lambda j: (0,)),
        ],
        out_specs=[
            pl.BlockSpec((BUF, _CB), lambda j: (0, j)),
            pl.BlockSpec((_CB,), lambda j: (j,)),
        ],
    )(M, u, v)


def _oute_body(a_ref, xl_ref, o_ref):
    o_ref[...] = jnp.dot(
        a_ref[...], xl_ref[...], preferred_element_type=jnp.float32
    ) * (1.0 / K)


def out_e_mm(A, Xl):
    return pl.pallas_call(
        _oute_body,
        out_shape=jax.ShapeDtypeStruct((BUF, FEAT), jnp.float32),
        grid=(BUF // _PB,),
        in_specs=[
            pl.BlockSpec((_PB, BUF), lambda i: (i, 0)),
            pl.BlockSpec((BUF, FEAT), lambda i: (0, 0)),
        ],
        out_specs=pl.BlockSpec((_PB, FEAT), lambda i: (i, 0)),
    )(A, Xl)


def _outn_body(a_ref, oe_ref, dn_ref, b_ref, o_ref):
    ob = jax.lax.dot_general(
        a_ref[...], oe_ref[...],
        dimension_numbers=(((0,), (0,)), ((), ())),
        preferred_element_type=jnp.float32,
    )
    o_ref[...] = ob * dn_ref[...][:, None] + b_ref[...][None, :]


def out_node_mm(A, out_e, dn, bias):
    return pl.pallas_call(
        _outn_body,
        out_shape=jax.ShapeDtypeStruct((BUF, FEAT), jnp.float32),
        grid=(BUF // _PB,),
        in_specs=[
            pl.BlockSpec((BUF, _PB), lambda j: (0, j)),
            pl.BlockSpec((BUF, FEAT), lambda j: (0, 0)),
            pl.BlockSpec((_PB,), lambda j: (j,)),
            pl.BlockSpec((FEAT,), lambda j: (0,)),
        ],
        out_specs=pl.BlockSpec((_PB, FEAT), lambda j: (j, 0)),
    )(A, out_e, dn, bias)


def _normfc_body(x_ref, gw_ref, gb_ref, gm_ref, wf_ref, bf_ref,
                 xn_ref, o1_ref):
    x = x_ref[...]
    mean = jnp.mean(x, axis=0, keepdims=True)
    cen = x - gm_ref[...][None, :] * mean
    var = jnp.mean(cen * cen, axis=0, keepdims=True)
    xn = cen / jnp.sqrt(var + 1e-5) * gw_ref[...][None, :] + gb_ref[...][None, :]
    xn = jnp.where(xn >= 0, xn, 0.01 * xn)
    xn_ref[...] = xn
    o1 = jnp.dot(xn, wf_ref[...], preferred_element_type=jnp.float32) + bf_ref[...][None, :]
    o1_ref[...] = jnp.where(o1 >= 0, o1, 0.01 * o1)


def norm_fc(x, gw, gb, gm, Wf, bf):
    hg = Wf.shape[1]
    return pl.pallas_call(
        _normfc_body,
        out_shape=[
            jax.ShapeDtypeStruct((BUF, FEAT), jnp.float32),
            jax.ShapeDtypeStruct((BUF, hg), jnp.float32),
        ],
        grid=(1,),
        in_specs=[
            pl.BlockSpec((BUF, FEAT), lambda i: (0, 0)),
            pl.BlockSpec((FEAT,), lambda i: (0,)),
            pl.BlockSpec((FEAT,), lambda i: (0,)),
            pl.BlockSpec((FEAT,), lambda i: (0,)),
            pl.BlockSpec((FEAT, hg), lambda i: (0, 0)),
            pl.BlockSpec((hg,), lambda i: (0,)),
        ],
        out_specs=[
            pl.BlockSpec((BUF, FEAT), lambda i: (0, 0)),
            pl.BlockSpec((BUF, hg), lambda i: (0, 0)),
        ],
    )(x, gw, gb, gm, Wf, bf)


_AB = 512


def _att_body(out_ref, w1_ref, b1_ref, w2_ref, b2_ref, s_ref):
    j = pl.program_id(0)
    T = jax.lax.dot_general(
        out_ref[...], w1_ref[...],
        dimension_numbers=(((0,), (0,)), ((), ())),
        preferred_element_type=jnp.float32,
    )
    T = jax.nn.relu(T + b1_ref[...][None, :])
    sp = jnp.sum(T * w2_ref[...][None, :], axis=1)

    @pl.when(j == 0)
    def _():
        s_ref[...] = sp + b2_ref[...]

    @pl.when(j != 0)
    def _():
        s_ref[...] += sp


def att_scores(out, Wat1, bat1, Wat2, bat2):
    D = out.shape[1]
    b2 = jnp.broadcast_to(bat2, (D,))
    return pl.pallas_call(
        _att_body,
        out_shape=jax.ShapeDtypeStruct((D,), jnp.float32),
        grid=(BUF // _AB,),
        in_specs=[
            pl.BlockSpec((BUF, D), lambda j: (0, 0)),
            pl.BlockSpec((BUF, _AB), lambda j: (0, j)),
            pl.BlockSpec((_AB,), lambda j: (j,)),
            pl.BlockSpec((_AB,), lambda j: (j,)),
            pl.BlockSpec((D,), lambda j: (0,)),
        ],
        out_specs=pl.BlockSpec((D,), lambda j: (0,)),
    )(out, Wat1, bat1, Wat2[:, 0], b2)


def _heads_body(s_ref, o_ref, wc1_ref, bc1_ref, wd_ref, bd_ref,
                wc2_ref, bc2_ref, wch_ref, bch_ref, d_ref, lg_ref):
    s = jax.nn.sigmoid(s_ref[...])
    s = s - jnp.mean(s)
    H = o_ref[...] * s[None, :]
    H1 = jnp.dot(H, wc1_ref[...], preferred_element_type=jnp.float32) + bc1_ref[...][None, :]
    H1 = jnp.where(H1 >= 0, H1, 0.01 * H1)
    d_ref[...] = jnp.dot(H1, wd_ref[...], preferred_element_type=jnp.float32) + bd_ref[...][None, :]
    H2 = jnp.dot(H1, wc2_ref[...], preferred_element_type=jnp.float32) + bc2_ref[...][None, :] + H1
    H2 = jnp.where(H2 >= 0, H2, 0.01 * H2)
    lg_ref[...] = jnp.dot(H2, wch_ref[...], preferred_element_type=jnp.float32) + bch_ref[...][None, :]


def heads(s, out20, g):
    hg = g['Wc2'].shape[0]
    D = out20.shape[1]
    o_pad = jnp.pad(out20, ((0, 32 - out20.shape[0]), (0, 0)))
    wd = jnp.pad(g['Wd'], ((0, 0), (0, 128 - NCLS)))
    bd = jnp.pad(g['bd'], (0, 128 - NCLS))
    wch = jnp.pad(g['Wch'], ((0, 0), (0, 128 - NCLS)))
    bch = jnp.pad(g['bch'], (0, 128 - NCLS))
    full = lambda shape: pl.BlockSpec(shape, lambda: tuple(0 for _ in shape))
    d, lg = pl.pallas_call(
        _heads_body,
        out_shape=[
            jax.ShapeDtypeStruct((32, 128), jnp.float32),
            jax.ShapeDtypeStruct((32, 128), jnp.float32),
        ],
        in_specs=[
            full((D,)), full((32, D)), full((D, hg)), full((hg,)),
            full((hg, 128)), full((128,)), full((hg, hg)), full((hg,)),
            full((hg, 128)), full((128,)),
        ],
        out_specs=[full((32, 128)), full((32, 128))],
    )(s, o_pad, g['Wc1'], g['bc1'], wd, bd, g['Wc2'], g['bc2'], wch, bch)
    return d[:BATCH, :NCLS], lg[:BATCH, :NCLS]


_EC = 125
_TILES = 16
_EPT = N_EDGES // _TILES
_NCH = _EPT // _EC
_NPAD = 10240
_RPT = _NPAD // _TILES
_HF = HID // 2


_LOG2E = 1.4426950408889634
_LN2_HI = 0.6931471824645996
_LN2_LO = -1.904654323148236e-9


def _gen_sc_body(xflat, srcr, dstr, tvec_h, zrows, out_h,
                 srcv, dstv, gbuf0, gbuf1, stg0, stg1, tv,
                 acc, gsem0, gsem1, ssem0, ssem1):
    cid = lax.axis_index("c")
    sid = lax.axis_index("s")
    pltpu.sync_copy(srcr.at[pl.ds(cid * (N_EDGES // _EC) + sid * _NCH, _NCH)],
                    srcv)
    pltpu.sync_copy(dstr.at[pl.ds(sid * _NCH, _NCH)], dstv)
    pltpu.sync_copy(tvec_h, tv)
    pltpu.sync_copy(zrows, acc.at[pl.ds(sid * _RPT, _RPT)])
    plsc.subcore_barrier()
    t16 = tv[...]

    def compute(gbuf, stg):
        def row(r, carry):
            for c in range(_HF // 16):
                v = gbuf[r, pl.ds(c * 16, 16)]
                m = jnp.maximum(v, 0.0) + 1e-7
                a = m * t16
                k = (a * _LOG2E + 12582912.0) - 12582912.0
                rr = (a - k * _LN2_HI) - k * _LN2_LO
                ki = jnp.clip(k, -126.0, 127.0).astype(jnp.int32)
                p2 = lax.bitcast_convert_type((ki + 127) << 23, jnp.float32)
                e = jnp.exp(rr) * p2
                stg[r, pl.ds(c * 16, 16)] = e
                stg[r, pl.ds(_HF + c * 16, 16)] = m * e
            return carry
        lax.fori_loop(0, _EC, row, 0, unroll=2)

    def chunk(k, carry):
        pltpu.async_copy(xflat.at[srcv.at[k]], gbuf0, gsem0).wait()
        compute(gbuf0, stg0)
        pltpu.async_copy(stg0, acc.at[dstv.at[k]], ssem0, add=True).wait()
        return carry

    lax.fori_loop(0, _NCH, chunk, 0)
    plsc.subcore_barrier()
    pltpu.sync_copy(acc.at[pl.ds(sid * _RPT, _RPT)],
                    out_h.at[pl.ds(cid * _NPAD + sid * _RPT, _RPT)])


def genconv_sc(x, src_off, dst_r, t):
    xflat = jnp.concatenate([x[:, :_HF], x[:, _HF:]], axis=0)
    tvec = jnp.broadcast_to(t.astype(jnp.float32).reshape(()), (16,))
    zrows = jnp.zeros((_RPT, HID), jnp.float32)
    mesh = plsc.VectorSubcoreMesh(core_axis_name="c", subcore_axis_name="s")
    acc_out = pl.kernel(
        _gen_sc_body,
        out_type=jax.ShapeDtypeStruct((2 * _NPAD, HID), jnp.float32),
        mesh=mesh,
        scratch_types=[
            pltpu.VMEM((_NCH, _EC), jnp.int32),
            pltpu.VMEM((_NCH, _EC), jnp.int32),
            pltpu.VMEM((_EC, _HF), jnp.float32),
            pltpu.VMEM((_EC, _HF), jnp.float32),
            pltpu.VMEM((_EC, HID), jnp.float32),
            pltpu.VMEM((_EC, HID), jnp.float32),
            pltpu.VMEM((16,), jnp.float32),
            pltpu.VMEM_SHARED((_NPAD, HID), jnp.float32),
            pltpu.SemaphoreType.DMA,
            pltpu.SemaphoreType.DMA,
            pltpu.SemaphoreType.DMA,
            pltpu.SemaphoreType.DMA,
        ],
        compiler_params=pltpu.CompilerParams(use_tc_tiling_on_sc=False),
    )(xflat, src_off, dst_r, tvec, zrows)
    den = jnp.concatenate([acc_out[:N_NODES, :_HF],
                           acc_out[_NPAD:_NPAD + N_NODES, :_HF]], axis=1)
    num = jnp.concatenate([acc_out[:N_NODES, _HF:],
                           acc_out[_NPAD:_NPAD + N_NODES, _HF:]], axis=1)
    return num, den


def _gcmlp_body(x_ref, num_ref, den_ref, w1_ref, b1_ref, g1_ref, bt1_ref,
                w2_ref, b2_ref, lg_ref, lb_ref, o_ref, *, post):
    x = x_ref[...]
    agg = num_ref[...] / (den_ref[...] + 1e-16) + x
    h = jnp.dot(agg, w1_ref[...], preferred_element_type=jnp.float32) + b1_ref[...][None, :]
    mu = jnp.mean(h, axis=1, keepdims=True)
    var = jnp.mean((h - mu) ** 2, axis=1, keepdims=True)
    h = (h - mu) / jnp.sqrt(var + 1e-5) * g1_ref[...][None, :] + bt1_ref[...][None, :]
    h = jax.nn.relu(h)
    o = jnp.dot(h, w2_ref[...], preferred_element_type=jnp.float32) + b2_ref[...][None, :]
    if post:
        mu2 = jnp.mean(o, axis=1, keepdims=True)
        var2 = jnp.mean((o - mu2) ** 2, axis=1, keepdims=True)
        o = (o - mu2) / jnp.sqrt(var2 + 1e-5) * lg_ref[...][None, :] + lb_ref[...][None, :]
        o = jax.nn.relu(o)
        o = x + o
    o_ref[...] = o


_GB = 1000


def gc_mlp(x, num, den, c, ln=None):
    post = ln is not None
    lg = ln['g'] if post else c['b2']
    lb = ln['b'] if post else c['b2']
    h2 = c['W1'].shape[1]
    body = functools.partial(_gcmlp_body, post=post)
    vec = lambda n: pl.BlockSpec((n,), lambda i: (0,))
    return pl.pallas_call(
        body,
        out_shape=jax.ShapeDtypeStruct((N_NODES, HID), jnp.float32),
        grid=(N_NODES // _GB,),
        in_specs=[
            pl.BlockSpec((_GB, HID), lambda i: (i, 0)),
            pl.BlockSpec((_GB, HID), lambda i: (i, 0)),
            pl.BlockSpec((_GB, HID), lambda i: (i, 0)),
            pl.BlockSpec((HID, h2), lambda i: (0, 0)),
            vec(h2),
            vec(h2),
            vec(h2),
            pl.BlockSpec((h2, HID), lambda i: (0, 0)),
            vec(HID),
            vec(HID),
            vec(HID),
        ],
        out_specs=pl.BlockSpec((_GB, HID), lambda i: (i, 0)),
    )(x, num, den, c['W1'], c['b1'], c['g1'], c['bt1'], c['W2'], c['b2'],
      lg, lb)


def genconv_fused(x, src_off, dst_r, c, ln=None):
    num, den = genconv_sc(x, src_off, dst_r, c['t'])
    return gc_mlp(x, num, den, c, ln)


def layer_norm(x, g, b, eps=1e-5):
    m = x.mean(-1, keepdims=True)
    v = ((x - m) ** 2).mean(-1, keepdims=True)
    return (x - m) / jnp.sqrt(v + eps) * g + b


def seg_softmax(vals, seg, num):
    m = jax.ops.segment_max(vals, seg, num)
    m = jnp.where(jnp.isneginf(m), 0.0, m)
    e = jnp.exp(vals - m[seg])
    s = jax.ops.segment_sum(e, seg, num)
    return e / (s[seg] + 1e-16)


def genconv(x, src, dst, c):
    msg = jax.nn.relu(x[src]) + 1e-7
    alpha = seg_softmax(msg * c['t'], dst, x.shape[0])
    out = jax.ops.segment_sum(msg * alpha, dst, x.shape[0])
    out = out + x
    h = out @ c['W1'] + c['b1']
    h = layer_norm(h, c['g1'], c['bt1'])
    h = jax.nn.relu(h)
    return h @ c['W2'] + c['b2']


def hypergraph_block(xc, params):
    g = params['gcn']
    M = topk_mask(xc)
    Xl1, u1, v1 = hyper_proj(xc, g['Whg1'], g['att1'])
    A1, dn1 = col_softmax(M, u1, v1)
    oute1 = out_e_mm(A1, Xl1)
    nx1 = out_node_mm(A1, oute1, dn1, g['bhg1'])
    xn1, out1 = norm_fc(nx1, g['gw1'], g['gb1'], g['gm1'], g['Wfc1'], g['bfc1'])
    Xl2, u2, v2 = hyper_proj(xn1, g['Whg2'], g['att2'], ea=xc)
    A2, dn2 = col_softmax(M, u2, v2)
    oute2 = out_e_mm(A2, Xl2)
    nx2 = out_node_mm(A2, oute2, dn2, g['bhg2'])
    xn2, out2 = norm_fc(nx2, g['gw2'], g['gb2'], g['gm2'], g['Wfc2'], g['bfc2'])

    out = jnp.concatenate([xc, out1, out2], axis=1)
    s = att_scores(out, g['Wat1'], g['bat1'], g['Wat2'], g['bat2'])
    return heads(s, out[:BATCH], g)


def kernel(x, edge_index, edge_latent, y, params):
    p = params
    src, dst = edge_index[0], edge_index[1]
    x = fc_relu(x, p['W_fc'], p['b_fc'])
    x_ = x
    x = genconv(x_, src, dst, p['convs'][0])
    x_ = jnp.concatenate([x_, x], axis=-1)
    for i in (1, 2):
        h = genconv(x, src, dst, p['convs'][i])
        h = layer_norm(h, p['lns'][i - 1]['g'], p['lns'][i - 1]['b'])
        h = jax.nn.relu(h)
        x = x + h
        x_ = jnp.concatenate([x_, x], axis=-1)
    h_path = x_.reshape(BATCH, 500, 4 * HID)
    h_path = jax.nn.relu(h_path @ p['Wphi'] + p['bphi'])
    a = jnp.tanh(h_path @ p['Wa'] + p['ba'])
    bgate = jax.nn.sigmoid(h_path @ p['Wb'] + p['bb'])
    A = (a * bgate) @ p['Wc'] + p['bc']
    A = jnp.swapaxes(A, -1, -2)
    h_path = jax.nn.softmax(A, axis=-1) @ h_path
    h = jax.nn.relu(h_path @ p['Wrho'] + p['brho'])[:, 0, :]
    logits = h @ p['Wcls'] + p['bcls']
    x_concat = jnp.concatenate([h, p['rehearsal']], axis=0)[:BUF]
    d, lg = hypergraph_block(x_concat, p)
    return logits, lg, d
